# Initial kernel scaffold; baseline (speedup 1.0000x reference)
import jax
import jax.numpy as jnp
from jax.experimental import pallas as pl


def kernel(input_y, sigma):
    raise NotImplementedError("write your pallas kernel here")


# speedup vs baseline: 1.0001x; 1.0001x over previous
import jax
import jax.numpy as jnp
import numpy as np
from jax.experimental import pallas as pl

P1, P2, M1, M2, WIN, STEP = 7, 7, 18, 55, 37, 4


def _subsample(x, s):
    if s == 1:
        return x
    N, C, H, W = x.shape
    rows = x[:, :, ::s, :]
    if H % s != 1:
        rows = jnp.concatenate([rows, x[:, :, -1:, :]], axis=2)
    out = rows[:, :, :, ::s]
    if W % s != 1:
        out = jnp.concatenate([out, rows[:, :, :, -1:]], axis=3)
    return out


def _sub_slice(x, i, j, Hc, Wc, s):
    body = x[:, :, i:i + Hc:s, :]
    if Hc % s != 1:
        body = jnp.concatenate([body, x[:, :, i + Hc - 1:i + Hc, :]], axis=2)
    out = body[:, :, :, j:j + Wc:s]
    if Wc % s != 1:
        out = jnp.concatenate([out, body[:, :, :, j + Wc - 1:j + Wc]], axis=3)
    return out


def _unfold(x, p):
    N, C, H, W = x.shape
    Ho, Wo = H - p + 1, W - p + 1
    cols = []
    for i in range(p):
        for j in range(p):
            cols.append(x[:, :, i:i + Ho, j:j + Wo])
    out = jnp.stack(cols, axis=2)
    return out.reshape(N, C * p * p, Ho * Wo)


def _fold(X, H, W, p):
    N, Cp2, L = X.shape
    C = Cp2 // (p * p)
    Ho, Wo = H - p + 1, W - p + 1
    Xr = X.reshape(N, C, p, p, Ho, Wo)
    out = jnp.zeros((N, C, H, W), X.dtype)
    for i in range(p):
        for j in range(p):
            out = out.at[:, :, i:i + Ho, j:j + Wo].add(Xr[:, :, i, j])
    return out


def _block_matching(input_x, m, p):
    input_x = jax.lax.stop_gradient(input_x)
    w, s = WIN, STEP
    N, C, H, W = input_x.shape
    r = w // 2
    Ho, Wo = H - p + 1, W - p + 1
    x_center = _unfold(input_x, p).reshape(N, C * p * p, Ho, Wo)
    x_pad = jnp.pad(x_center, ((0, 0), (0, 0), (r, r), (r, r)),
                    mode='constant', constant_values=jnp.inf)
    xc = _subsample(x_center, s)
    dists = []
    for i in range(w):
        for j in range(w):
            x_ij = _sub_slice(x_pad, i, j, Ho, Wo, s)
            dists.append(jnp.sum((x_ij - xc) ** 2, axis=1))
    x_dist = jnp.stack(dists, axis=1)
    x_dist = x_dist.at[:, r * w + r, :, :].set(-jnp.inf)
    d = jnp.moveaxis(x_dist, 1, -1)
    _, idx = jax.lax.top_k(-d, m)
    indices = jnp.moveaxis(idx, -1, 1)
    ind_rows = indices // w - r
    ind_cols = indices % w - r
    row_full = jnp.broadcast_to(jnp.arange(r, Ho + r)[None, None, :, None], (N, m, Ho, Wo))
    col_full = jnp.broadcast_to(jnp.arange(r, Wo + r)[None, None, None, :], (N, m, Ho, Wo))
    row_ar = _subsample(row_full, s)
    col_ar = _subsample(col_full, s)
    indices_row = ind_rows + row_ar - r
    indices_col = ind_cols + col_ar - r
    flat = indices_row * Wo + indices_col
    flat = flat.reshape(N, m, -1)
    flat = jnp.swapaxes(flat, 1, 2).reshape(N, -1)
    return flat


def _group_patches(input_y, indices, m, n, p):
    uf = _unfold(input_y, p)
    N = uf.shape[0]
    K = indices.shape[1]
    idx = jnp.broadcast_to(indices[:, None, :], (N, n, K))
    Y = jnp.take_along_axis(uf, idx, axis=2)
    Y = jnp.swapaxes(Y, 1, 2)
    return Y.reshape(N, K // m, m, n)


def _aggregation(X_hat, weights, indices, input_y, p):
    N, C, H, W = input_y.shape
    n = X_hat.shape[3]
    L = (H - p + 1) * (W - p + 1)
    Xw = X_hat * weights
    Xw = jnp.transpose(Xw, (0, 3, 1, 2)).reshape(N, n, -1)
    wflat = weights.reshape(N, -1)
    xs_list, dv_list = [], []
    for i in range(N):
        xs = jax.ops.segment_sum(Xw[i].T, indices[i], num_segments=L).T
        dv = jax.ops.segment_sum(wflat[i], indices[i], num_segments=L)
        xs_list.append(xs)
        dv_list.append(jnp.broadcast_to(dv[None, :], (n, L)))
    X_sum = jnp.stack(xs_list, axis=0)
    divisor = jnp.stack(dv_list, axis=0)
    num = _fold(X_sum, H, W, p)
    den = _fold(divisor, H, W, p)
    return num / den


def _denoise1(Y, sigma):
    N, B, m, n = Y.shape
    YtY = Y @ jnp.swapaxes(Y, 2, 3)
    Im = jnp.eye(m, dtype=Y.dtype)
    theta = jnp.linalg.solve(YtY, YtY - n * sigma ** 2 * Im)
    theta = jnp.swapaxes(theta, 2, 3)
    X_hat = theta @ Y
    weights = 1.0 / jnp.sum(theta ** 2, axis=3, keepdims=True)
    return X_hat, weights


def _denoise2(Y, X, sigma):
    N, B, m, n = Y.shape
    XtX = X @ jnp.swapaxes(X, 2, 3)
    Im = jnp.eye(m, dtype=Y.dtype)
    theta = jnp.linalg.solve(XtX + n * sigma ** 2 * Im, XtX)
    theta = jnp.swapaxes(theta, 2, 3)
    X_hat = theta @ Y
    weights = 1.0 / jnp.sum(theta ** 2, axis=3, keepdims=True)
    return X_hat, weights


def _identity_kernel(x_ref, o_ref):
    o_ref[...] = x_ref[...]


def _pallas_identity(x):
    return pl.pallas_call(
        _identity_kernel,
        out_shape=jax.ShapeDtypeStruct(x.shape, x.dtype),
    )(x)


def _step1(input_y, sigma):
    C = input_y.shape[1]
    p, m = P1, M1
    y_block = jnp.mean(input_y, axis=1, keepdims=True)
    indices = _block_matching(y_block, m, p)
    Y = _group_patches(input_y, indices, m, C * p * p, p)
    X_hat, weights = _denoise1(Y, sigma)
    return _aggregation(X_hat, weights, indices, input_y, p)


def _step2(input_y, input_x, sigma):
    C = input_y.shape[1]
    p, m = P2, M2
    x_block = jnp.mean(input_x, axis=1, keepdims=True)
    indices = _block_matching(x_block, m, p)
    Y = _group_patches(input_y, indices, m, C * p * p, p)
    X = _group_patches(input_x, indices, m, C * p * p, p)
    X_hat, weights = _denoise2(Y, X, sigma)
    return _aggregation(X_hat, weights, indices, input_y, p)


def kernel(input_y, sigma):
    input_y = _pallas_identity(input_y)
    den1 = _step1(input_y, sigma)
    return _step2(input_y, den1, sigma)


# speedup vs baseline: 1.1151x; 1.1151x over previous
import jax
import jax.numpy as jnp
import numpy as np
from jax.experimental import pallas as pl

P1, P2, M1, M2, WIN, STEP = 7, 7, 18, 55, 37, 4


def _subsample(x, s):
    if s == 1:
        return x
    N, C, H, W = x.shape
    rows = x[:, :, ::s, :]
    if H % s != 1:
        rows = jnp.concatenate([rows, x[:, :, -1:, :]], axis=2)
    out = rows[:, :, :, ::s]
    if W % s != 1:
        out = jnp.concatenate([out, rows[:, :, :, -1:]], axis=3)
    return out


def _unfold(x, p):
    N, C, H, W = x.shape
    Ho, Wo = H - p + 1, W - p + 1
    cols = []
    for i in range(p):
        for j in range(p):
            cols.append(x[:, :, i:i + Ho, j:j + Wo])
    out = jnp.stack(cols, axis=2)
    return out.reshape(N, C * p * p, Ho * Wo)


def _fold(X, H, W, p):
    N, Cp2, L = X.shape
    C = Cp2 // (p * p)
    Ho, Wo = H - p + 1, W - p + 1
    Xr = X.reshape(N, C, p, p, Ho, Wo)
    out = jnp.zeros((N, C, H, W), X.dtype)
    for i in range(p):
        for j in range(p):
            out = out.at[:, :, i:i + Ho, j:j + Wo].add(Xr[:, :, i, j])
    return out


def _dist_pallas(x_pad, xc):
    C2, Hp, _ = x_pad.shape
    w, s = WIN, STEP
    Ho = Hp - 2 * (w // 2)
    nq = xc.shape[1]
    qpos = list(range(0, Ho - 1, s)) + [Ho - 1]

    band = jnp.stack(
        [jax.lax.slice(x_pad, (0, q, 0), (C2, q + w, Hp)) for q in qpos],
        axis=0)
    band = jnp.transpose(band, (2, 0, 1, 3))
    xct = jnp.transpose(xc, (1, 0, 2))

    def kern(b_ref, xc_ref, o_ref):
        R = b_ref[0]
        c = xc_ref[...]
        for b in range(nq):
            diff = R - c[:, :, b:b + 1]
            o_ref[0, :, b] = jnp.sum(diff * diff, axis=1)

    D = pl.pallas_call(
        kern,
        grid=(w,),
        in_specs=[
            pl.BlockSpec((1, nq, C2, Hp), lambda i: (i, 0, 0, 0)),
            pl.BlockSpec((nq, C2, nq), lambda i: (0, 0, 0)),
        ],
        out_specs=pl.BlockSpec((1, nq, nq, Hp), lambda i: (i, 0, 0, 0)),
        out_shape=jax.ShapeDtypeStruct((w, nq, nq, Hp), x_pad.dtype),
    )(band, xct)
    idx = (jnp.asarray(qpos, jnp.int32)[:, None]
           + jnp.arange(w, dtype=jnp.int32)[None, :])
    sel = jnp.take_along_axis(
        D, jnp.broadcast_to(idx[None, None], (w, nq, nq, w)), axis=3)
    return jnp.transpose(sel, (0, 3, 1, 2))


def _block_matching(input_x, m, p):
    input_x = jax.lax.stop_gradient(input_x)
    w, s = WIN, STEP
    N, C, H, W = input_x.shape
    r = w // 2
    Ho, Wo = H - p + 1, W - p + 1
    x_center = _unfold(input_x, p).reshape(N, C * p * p, Ho, Wo)
    x_pad = jnp.pad(x_center, ((0, 0), (0, 0), (r, r), (r, r)),
                    mode='constant', constant_values=jnp.inf)
    xc = _subsample(x_center, s)
    per_n = [_dist_pallas(x_pad[n], xc[n]) for n in range(N)]
    nq = xc.shape[2]
    x_dist = jnp.stack(per_n, axis=0).reshape(N, w * w, nq, nq)
    x_dist = x_dist.at[:, r * w + r, :, :].set(-jnp.inf)
    d = jnp.moveaxis(x_dist, 1, -1)
    _, idx = jax.lax.top_k(-d, m)
    indices = jnp.moveaxis(idx, -1, 1)
    ind_rows = indices // w - r
    ind_cols = indices % w - r
    row_full = jnp.broadcast_to(jnp.arange(r, Ho + r)[None, None, :, None], (N, m, Ho, Wo))
    col_full = jnp.broadcast_to(jnp.arange(r, Wo + r)[None, None, None, :], (N, m, Ho, Wo))
    row_ar = _subsample(row_full, s)
    col_ar = _subsample(col_full, s)
    indices_row = ind_rows + row_ar - r
    indices_col = ind_cols + col_ar - r
    flat = indices_row * Wo + indices_col
    flat = flat.reshape(N, m, -1)
    flat = jnp.swapaxes(flat, 1, 2).reshape(N, -1)
    return flat


def _group_patches(input_y, indices, m, n, p):
    uf = _unfold(input_y, p)
    N = uf.shape[0]
    K = indices.shape[1]
    idx = jnp.broadcast_to(indices[:, None, :], (N, n, K))
    Y = jnp.take_along_axis(uf, idx, axis=2)
    Y = jnp.swapaxes(Y, 1, 2)
    return Y.reshape(N, K // m, m, n)


def _aggregation(X_hat, weights, indices, input_y, p):
    N, C, H, W = input_y.shape
    n = X_hat.shape[3]
    L = (H - p + 1) * (W - p + 1)
    Xw = X_hat * weights
    Xw = jnp.transpose(Xw, (0, 3, 1, 2)).reshape(N, n, -1)
    wflat = weights.reshape(N, -1)
    xs_list, dv_list = [], []
    for i in range(N):
        xs = jax.ops.segment_sum(Xw[i].T, indices[i], num_segments=L).T
        dv = jax.ops.segment_sum(wflat[i], indices[i], num_segments=L)
        xs_list.append(xs)
        dv_list.append(jnp.broadcast_to(dv[None, :], (n, L)))
    X_sum = jnp.stack(xs_list, axis=0)
    divisor = jnp.stack(dv_list, axis=0)
    num = _fold(X_sum, H, W, p)
    den = _fold(divisor, H, W, p)
    return num / den


def _denoise1(Y, sigma):
    N, B, m, n = Y.shape
    YtY = Y @ jnp.swapaxes(Y, 2, 3)
    Im = jnp.eye(m, dtype=Y.dtype)
    theta = jnp.linalg.solve(YtY, YtY - n * sigma ** 2 * Im)
    theta = jnp.swapaxes(theta, 2, 3)
    X_hat = theta @ Y
    weights = 1.0 / jnp.sum(theta ** 2, axis=3, keepdims=True)
    return X_hat, weights


def _denoise2(Y, X, sigma):
    N, B, m, n = Y.shape
    XtX = X @ jnp.swapaxes(X, 2, 3)
    Im = jnp.eye(m, dtype=Y.dtype)
    theta = jnp.linalg.solve(XtX + n * sigma ** 2 * Im, XtX)
    theta = jnp.swapaxes(theta, 2, 3)
    X_hat = theta @ Y
    weights = 1.0 / jnp.sum(theta ** 2, axis=3, keepdims=True)
    return X_hat, weights


def _step1(input_y, sigma):
    C = input_y.shape[1]
    p, m = P1, M1
    y_block = jnp.mean(input_y, axis=1, keepdims=True)
    indices = _block_matching(y_block, m, p)
    Y = _group_patches(input_y, indices, m, C * p * p, p)
    X_hat, weights = _denoise1(Y, sigma)
    return _aggregation(X_hat, weights, indices, input_y, p)


def _step2(input_y, input_x, sigma):
    C = input_y.shape[1]
    p, m = P2, M2
    x_block = jnp.mean(input_x, axis=1, keepdims=True)
    indices = _block_matching(x_block, m, p)
    Y = _group_patches(input_y, indices, m, C * p * p, p)
    X = _group_patches(input_x, indices, m, C * p * p, p)
    X_hat, weights = _denoise2(Y, X, sigma)
    return _aggregation(X_hat, weights, indices, input_y, p)


def kernel(input_y, sigma):
    den1 = _step1(input_y, sigma)
    return _step2(input_y, den1, sigma)


# speedup vs baseline: 1.1992x; 1.0754x over previous
import jax
import jax.numpy as jnp
import numpy as np
from jax.experimental import pallas as pl

P1, P2, M1, M2, WIN, STEP = 7, 7, 18, 55, 37, 4


def _subsample(x, s):
    if s == 1:
        return x
    N, C, H, W = x.shape
    rows = x[:, :, ::s, :]
    if H % s != 1:
        rows = jnp.concatenate([rows, x[:, :, -1:, :]], axis=2)
    out = rows[:, :, :, ::s]
    if W % s != 1:
        out = jnp.concatenate([out, rows[:, :, :, -1:]], axis=3)
    return out


def _unfold(x, p):
    N, C, H, W = x.shape
    Ho, Wo = H - p + 1, W - p + 1
    cols = []
    for i in range(p):
        for j in range(p):
            cols.append(x[:, :, i:i + Ho, j:j + Wo])
    out = jnp.stack(cols, axis=2)
    return out.reshape(N, C * p * p, Ho * Wo)


def _fold(X, H, W, p):
    N, Cp2, L = X.shape
    C = Cp2 // (p * p)
    Ho, Wo = H - p + 1, W - p + 1
    Xr = X.reshape(N, C, p, p, Ho, Wo)
    out = jnp.zeros((N, C, H, W), X.dtype)
    for i in range(p):
        for j in range(p):
            out = out.at[:, :, i:i + Ho, j:j + Wo].add(Xr[:, :, i, j])
    return out


def _dist_pallas(x_pad, xc):
    C2, Hp, _ = x_pad.shape
    w, s = WIN, STEP
    Ho = Hp - 2 * (w // 2)
    nq = xc.shape[1]
    qpos = list(range(0, Ho - 1, s)) + [Ho - 1]

    band = jnp.stack(
        [jax.lax.slice(x_pad, (0, q, 0), (C2, q + w, Hp)) for q in qpos],
        axis=0)
    band = jnp.transpose(band, (2, 0, 1, 3))
    xct = jnp.transpose(xc, (1, 0, 2))

    def kern(b_ref, xc_ref, o_ref):
        R = b_ref[0]
        c = xc_ref[...]
        for b in range(nq):
            diff = R - c[:, :, b:b + 1]
            o_ref[0, :, b] = jnp.sum(diff * diff, axis=1)

    D = pl.pallas_call(
        kern,
        grid=(w,),
        in_specs=[
            pl.BlockSpec((1, nq, C2, Hp), lambda i: (i, 0, 0, 0)),
            pl.BlockSpec((nq, C2, nq), lambda i: (0, 0, 0)),
        ],
        out_specs=pl.BlockSpec((1, nq, nq, Hp), lambda i: (i, 0, 0, 0)),
        out_shape=jax.ShapeDtypeStruct((w, nq, nq, Hp), x_pad.dtype),
    )(band, xct)
    idx = (jnp.asarray(qpos, jnp.int32)[:, None]
           + jnp.arange(w, dtype=jnp.int32)[None, :])
    sel = jnp.take_along_axis(
        D, jnp.broadcast_to(idx[None, None], (w, nq, nq, w)), axis=3)
    return jnp.transpose(sel, (0, 3, 1, 2))


def _block_matching(input_x, m, p):
    input_x = jax.lax.stop_gradient(input_x)
    w, s = WIN, STEP
    N, C, H, W = input_x.shape
    r = w // 2
    Ho, Wo = H - p + 1, W - p + 1
    x_center = _unfold(input_x, p).reshape(N, C * p * p, Ho, Wo)
    x_pad = jnp.pad(x_center, ((0, 0), (0, 0), (r, r), (r, r)),
                    mode='constant', constant_values=jnp.inf)
    xc = _subsample(x_center, s)
    per_n = [_dist_pallas(x_pad[n], xc[n]) for n in range(N)]
    nq = xc.shape[2]
    x_dist = jnp.stack(per_n, axis=0).reshape(N, w * w, nq, nq)
    x_dist = x_dist.at[:, r * w + r, :, :].set(-jnp.inf)
    d = jnp.moveaxis(x_dist, 1, -1)
    _, idx = jax.lax.top_k(-d, m)
    indices = jnp.moveaxis(idx, -1, 1)
    ind_rows = indices // w - r
    ind_cols = indices % w - r
    row_full = jnp.broadcast_to(jnp.arange(r, Ho + r)[None, None, :, None], (N, m, Ho, Wo))
    col_full = jnp.broadcast_to(jnp.arange(r, Wo + r)[None, None, None, :], (N, m, Ho, Wo))
    row_ar = _subsample(row_full, s)
    col_ar = _subsample(col_full, s)
    indices_row = ind_rows + row_ar - r
    indices_col = ind_cols + col_ar - r
    flat = indices_row * Wo + indices_col
    flat = flat.reshape(N, m, -1)
    flat = jnp.swapaxes(flat, 1, 2).reshape(N, -1)
    return flat


def _group_patches(input_y, indices, m, n, p):
    uf = _unfold(input_y, p)
    N = uf.shape[0]
    K = indices.shape[1]
    idx = jnp.broadcast_to(indices[:, None, :], (N, n, K))
    Y = jnp.take_along_axis(uf, idx, axis=2)
    Y = jnp.swapaxes(Y, 1, 2)
    return Y.reshape(N, K // m, m, n)


def _aggregation(X_hat, weights, indices, input_y, p):
    N, C, H, W = input_y.shape
    n = X_hat.shape[3]
    L = (H - p + 1) * (W - p + 1)
    Xw = X_hat * weights
    Xw = jnp.transpose(Xw, (0, 3, 1, 2)).reshape(N, n, -1)
    wflat = weights.reshape(N, -1)
    xs_list, dv_list = [], []
    for i in range(N):
        xs = jax.ops.segment_sum(Xw[i].T, indices[i], num_segments=L).T
        dv = jax.ops.segment_sum(wflat[i], indices[i], num_segments=L)
        xs_list.append(xs)
        dv_list.append(jnp.broadcast_to(dv[None, :], (n, L)))
    X_sum = jnp.stack(xs_list, axis=0)
    divisor = jnp.stack(dv_list, axis=0)
    num = _fold(X_sum, H, W, p)
    den = _fold(divisor, H, W, p)
    return num / den


def _solve_pallas(A, B):
    bt, m, _ = A.shape
    bs = 64 if bt % 64 == 0 else bt

    def kern(a_ref, b_ref, o_ref):
        A_ = a_ref[...]
        B_ = b_ref[...]
        rows = jax.lax.broadcasted_iota(jnp.int32, (m, m), 0)
        for k in range(m):
            piv = A_[:, k:k + 1, k:k + 1]
            ak = A_[:, k:k + 1, :] / piv
            bk = B_[:, k:k + 1, :] / piv
            fa = A_[:, :, k:k + 1]
            fa = jnp.where((rows == k)[None, :, k:k + 1], 0.0, fa)
            A_ = A_ - fa * ak
            B_ = B_ - fa * bk
            A_ = jnp.where((rows == k)[None], ak, A_)
            B_ = jnp.where((rows == k)[None], bk, B_)
        o_ref[...] = B_

    return pl.pallas_call(
        kern,
        grid=(bt // bs,),
        in_specs=[
            pl.BlockSpec((bs, m, m), lambda i: (i, 0, 0)),
            pl.BlockSpec((bs, m, m), lambda i: (i, 0, 0)),
        ],
        out_specs=pl.BlockSpec((bs, m, m), lambda i: (i, 0, 0)),
        out_shape=jax.ShapeDtypeStruct((bt, m, m), A.dtype),
    )(A, B)


def _batch_solve(A, B):
    N, bt, m, _ = A.shape
    out = [_solve_pallas(A[n], B[n]) for n in range(N)]
    return jnp.stack(out, axis=0)


def _denoise1(Y, sigma):
    N, B, m, n = Y.shape
    YtY = Y @ jnp.swapaxes(Y, 2, 3)
    Im = jnp.eye(m, dtype=Y.dtype)
    theta = _batch_solve(YtY, YtY - n * sigma ** 2 * Im)
    theta = jnp.swapaxes(theta, 2, 3)
    X_hat = theta @ Y
    weights = 1.0 / jnp.sum(theta ** 2, axis=3, keepdims=True)
    return X_hat, weights


def _denoise2(Y, X, sigma):
    N, B, m, n = Y.shape
    XtX = X @ jnp.swapaxes(X, 2, 3)
    Im = jnp.eye(m, dtype=Y.dtype)
    theta = _batch_solve(XtX + n * sigma ** 2 * Im, XtX)
    theta = jnp.swapaxes(theta, 2, 3)
    X_hat = theta @ Y
    weights = 1.0 / jnp.sum(theta ** 2, axis=3, keepdims=True)
    return X_hat, weights


def _step1(input_y, sigma):
    C = input_y.shape[1]
    p, m = P1, M1
    y_block = jnp.mean(input_y, axis=1, keepdims=True)
    indices = _block_matching(y_block, m, p)
    Y = _group_patches(input_y, indices, m, C * p * p, p)
    X_hat, weights = _denoise1(Y, sigma)
    return _aggregation(X_hat, weights, indices, input_y, p)


def _step2(input_y, input_x, sigma):
    C = input_y.shape[1]
    p, m = P2, M2
    x_block = jnp.mean(input_x, axis=1, keepdims=True)
    indices = _block_matching(x_block, m, p)
    Y = _group_patches(input_y, indices, m, C * p * p, p)
    X = _group_patches(input_x, indices, m, C * p * p, p)
    X_hat, weights = _denoise2(Y, X, sigma)
    return _aggregation(X_hat, weights, indices, input_y, p)


def kernel(input_y, sigma):
    den1 = _step1(input_y, sigma)
    return _step2(input_y, den1, sigma)
